# baseline (speedup 1.0000x reference)
import functools

import jax
import jax.numpy as jnp
from jax import lax
from jax.experimental import pallas as pl
from jax.experimental.pallas import tpu as pltpu
from jax.experimental.pallas import tpu_sc as plsc

_N_CTX = 4
_SEQ = 77
_DIM = 512
_IDXROW = 80
_MAIN = 72
_TAIL = _SEQ - _MAIN
_LANES = 16


def _sc_prompt_gather(idx_flat, table, ctx_flat, n_cls):
    info = plsc.get_sparse_core_info()
    nw = info.num_cores * info.num_subcores
    rpw = idx_flat.shape[0] // (nw * _IDXROW)
    nfull = n_cls // nw
    rem = n_cls % nw
    mesh = plsc.VectorSubcoreMesh(core_axis_name="c", subcore_axis_name="s")

    @functools.partial(
        pl.kernel,
        mesh=mesh,
        out_type=jax.ShapeDtypeStruct((n_cls, _SEQ, _DIM), jnp.float32),
        scratch_types=[
            pltpu.VMEM((rpw * _IDXROW,), jnp.int32),
            pltpu.VMEM((_N_CTX * _DIM,), jnp.float32),
            pltpu.VMEM((_SEQ, _DIM), jnp.float32),
            pltpu.VMEM((_SEQ, _DIM), jnp.float32),
            pltpu.VMEM((8, _DIM), jnp.float32),
            pltpu.VMEM((8, _DIM), jnp.float32),
            pltpu.SemaphoreType.DMA,
            pltpu.SemaphoreType.DMA,
        ],
    )
    def k(idx_hbm, table_hbm, ctx_hbm, out_hbm,
          idx_v, ctx_v, row0, row1, tb0, tb1, gs0, gs1):
        wid = lax.axis_index("s") * info.num_cores + lax.axis_index("c")
        nr = nfull + (wid < rem).astype(jnp.int32)

        pltpu.sync_copy(idx_hbm.at[pl.ds(wid * (rpw * _IDXROW),
                                         rpw * _IDXROW)], idx_v)
        pltpu.sync_copy(ctx_hbm, ctx_v)

        def issue_gather(r, buf, tbuf, sem):
            pltpu.async_copy(
                table_hbm.at[idx_v.at[pl.ds(r * _IDXROW, _MAIN)]],
                buf.at[pl.ds(0, _MAIN)], sem)
            pltpu.async_copy(
                table_hbm.at[idx_v.at[pl.ds(r * _IDXROW + _MAIN, 8)]],
                tbuf, sem)

        def wait_gather(buf, tbuf, sem):
            pltpu.make_async_copy(
                table_hbm.at[idx_v.at[pl.ds(0, _MAIN)]],
                buf.at[pl.ds(0, _MAIN)], sem).wait()
            pltpu.make_async_copy(
                table_hbm.at[idx_v.at[pl.ds(_MAIN, 8)]], tbuf, sem).wait()

        def round_sect(r, buf, tbuf, sem, obuf, otbuf, osem):
            wait_gather(buf, tbuf, sem)

            @pl.when(r + 1 < nr)
            def _():
                issue_gather(r + 1, obuf, otbuf, osem)

            for j in range(_TAIL):
                for i in range(_DIM // _LANES):
                    buf[_MAIN + j, pl.ds(i * _LANES, _LANES)] = (
                        tbuf[j, pl.ds(i * _LANES, _LANES)])
            for j in range(_N_CTX):
                for i in range(_DIM // _LANES):
                    buf[1 + j, pl.ds(i * _LANES, _LANES)] = (
                        ctx_v[pl.ds(j * _DIM + i * _LANES, _LANES)])

            c = r * nw + wid
            pltpu.sync_copy(buf, out_hbm.at[c])

        issue_gather(0, row0, tb0, gs0)

        def body(rr, _):
            r0 = rr * 2
            round_sect(r0, row0, tb0, gs0, row1, tb1, gs1)

            @pl.when(r0 + 1 < nr)
            def _():
                round_sect(r0 + 1, row1, tb1, gs1, row0, tb0, gs0)

            return _

        lax.fori_loop(0, (rpw + 1) // 2, body, None)

    return k(idx_flat, table, ctx_flat)


def kernel(tokenized_prompts, token_embedding, ctx):
    n_cls = tokenized_prompts.shape[0]
    info = plsc.get_sparse_core_info()
    nw = info.num_cores * info.num_subcores
    rpw = -(-n_cls // nw)
    pad = nw * rpw - n_cls

    tokp = jnp.concatenate(
        [tokenized_prompts,
         jnp.zeros((pad, tokenized_prompts.shape[1]), jnp.int32)], axis=0)
    by_worker = tokp.reshape(rpw, nw, _SEQ).transpose(1, 0, 2)
    idx_flat = jnp.concatenate(
        [by_worker, jnp.zeros((nw, rpw, _IDXROW - _SEQ), jnp.int32)],
        axis=2).reshape(-1)

    return _sc_prompt_gather(idx_flat, token_embedding, ctx.reshape(-1),
                             n_cls)
